# baseline (speedup 1.0000x reference)
import functools

import jax
import jax.numpy as jnp
from jax import lax
from jax.experimental import pallas as pl
from jax.experimental.pallas import tpu as pltpu
from jax.experimental.pallas import tpu_sc as plsc

NC = 2
NS = 16
NW = NC * NS

ROWS = 16384
COLS = 26
DIM = 64
B = ROWS * COLS
CH = 128
NCH = B // (NW * CH)
assert NW * NCH * CH == B


def _gather_body(idx_hbm, table_hbm, out_hbm, idx_v, rows_v, sem):
    wid = lax.axis_index("s") * NC + lax.axis_index("c")
    base = wid * (NCH * CH)
    pltpu.sync_copy(idx_hbm.at[wid], idx_v)

    def step(j, carry):
        pltpu.async_copy(table_hbm.at[idx_v.at[j]], rows_v, sem).wait()
        pltpu.sync_copy(rows_v, out_hbm.at[pl.ds(base + j * CH, CH)])
        return carry

    lax.fori_loop(0, NCH, step, 0)


@jax.jit
def _gather(idx3, table):
    mesh = plsc.VectorSubcoreMesh(core_axis_name="c", subcore_axis_name="s")
    return pl.kernel(
        _gather_body,
        out_type=jax.ShapeDtypeStruct((B, DIM), jnp.float32),
        mesh=mesh,
        scratch_types=[
            pltpu.VMEM((NCH, CH), jnp.int32),
            pltpu.VMEM((CH, DIM), jnp.float32),
            pltpu.SemaphoreType.DMA,
        ],
        compiler_params=pltpu.CompilerParams(use_tc_tiling_on_sc=False),
    )(idx3, table)


def kernel(attributes, table):
    shape = attributes.shape
    idx3 = attributes.astype(jnp.int32).reshape(NW, NCH, CH)
    out = _gather(idx3, table)
    return out.reshape(shape + (DIM,))


# speedup vs baseline: 1.0570x; 1.0570x over previous
import functools

import jax
import jax.numpy as jnp
from jax import lax
from jax.experimental import pallas as pl
from jax.experimental.pallas import tpu as pltpu
from jax.experimental.pallas import tpu_sc as plsc

NC = 2
NS = 16
NW = NC * NS

ROWS = 16384
COLS = 26
DIM = 64
B = ROWS * COLS
CH = 512
NCH = B // (NW * CH)
assert NW * NCH * CH == B


def _gather_body(idx_hbm, table_hbm, out_hbm, idx_v, rows_v, sem):
    wid = lax.axis_index("s") * NC + lax.axis_index("c")
    base = wid * (NCH * CH)
    pltpu.sync_copy(idx_hbm.at[wid], idx_v)

    def step(j, carry):
        pltpu.async_copy(table_hbm.at[idx_v.at[j]], rows_v, sem).wait()
        pltpu.sync_copy(rows_v, out_hbm.at[pl.ds(base + j * CH, CH)])
        return carry

    lax.fori_loop(0, NCH, step, 0)


@jax.jit
def _gather(idx3, table):
    mesh = plsc.VectorSubcoreMesh(core_axis_name="c", subcore_axis_name="s")
    return pl.kernel(
        _gather_body,
        out_type=jax.ShapeDtypeStruct((B, DIM), jnp.float32),
        mesh=mesh,
        scratch_types=[
            pltpu.VMEM((NCH, CH), jnp.int32),
            pltpu.VMEM((CH, DIM), jnp.float32),
            pltpu.SemaphoreType.DMA,
        ],
        compiler_params=pltpu.CompilerParams(use_tc_tiling_on_sc=False),
    )(idx3, table)


def kernel(attributes, table):
    shape = attributes.shape
    idx3 = attributes.astype(jnp.int32).reshape(NW, NCH, CH)
    out = _gather(idx3, table)
    return out.reshape(shape + (DIM,))


# speedup vs baseline: 1.0694x; 1.0117x over previous
import jax
import jax.numpy as jnp
from jax import lax
from jax.experimental import pallas as pl
from jax.experimental.pallas import tpu as pltpu
from jax.experimental.pallas import tpu_sc as plsc

NC = 2
NS = 16
NW = NC * NS

ROWS = 16384
COLS = 26
DIM = 64
B = ROWS * COLS
CH = 256
NBUF = 4
NCH = B // (NW * CH)
G = NCH // NBUF
assert NW * NCH * CH == B and NCH % NBUF == 0


def _gather_body(idx_hbm, table_hbm, out_hbm, idx_v, *rest):
    rows_v = rest[:NBUF]
    gsem = rest[NBUF:2 * NBUF]
    osem = rest[2 * NBUF:3 * NBUF]
    wid = lax.axis_index("s") * NC + lax.axis_index("c")
    base = wid * (NCH * CH)
    pltpu.sync_copy(idx_hbm.at[wid], idx_v)

    def out_slice(j):
        return out_hbm.at[pl.ds(base + j * CH, CH)]

    descs = [
        pltpu.async_copy(table_hbm.at[idx_v.at[b]], rows_v[b], gsem[b])
        for b in range(NBUF)
    ]
    for b in range(NBUF):
        descs[b].wait()
        pltpu.async_copy(rows_v[b], out_slice(b), osem[b])

    def body(g, carry):
        descs = []
        for b in range(NBUF):
            j = g * NBUF + b
            pltpu.make_async_copy(rows_v[b], out_slice(j - NBUF), osem[b]).wait()
            descs.append(
                pltpu.async_copy(table_hbm.at[idx_v.at[j]], rows_v[b], gsem[b]))
        for b in range(NBUF):
            j = g * NBUF + b
            descs[b].wait()
            pltpu.async_copy(rows_v[b], out_slice(j), osem[b])
        return carry

    lax.fori_loop(1, G, body, 0)

    for b in range(NBUF):
        j = (G - 1) * NBUF + b
        pltpu.make_async_copy(rows_v[b], out_slice(j), osem[b]).wait()


@jax.jit
def _gather(idx3, table):
    mesh = plsc.VectorSubcoreMesh(core_axis_name="c", subcore_axis_name="s")
    return pl.kernel(
        _gather_body,
        out_type=jax.ShapeDtypeStruct((B, DIM), jnp.float32),
        mesh=mesh,
        scratch_types=(
            [pltpu.VMEM((NCH, CH), jnp.int32)]
            + [pltpu.VMEM((CH, DIM), jnp.float32) for _ in range(NBUF)]
            + [pltpu.SemaphoreType.DMA for _ in range(2 * NBUF)]
        ),
        compiler_params=pltpu.CompilerParams(use_tc_tiling_on_sc=False),
    )(idx3, table)


def kernel(attributes, table):
    shape = attributes.shape
    idx3 = attributes.astype(jnp.int32).reshape(NW, NCH, CH)
    out = _gather(idx3, table)
    return out.reshape(shape + (DIM,))


# speedup vs baseline: 1.0877x; 1.0172x over previous
import jax
import jax.numpy as jnp
from jax import lax
from jax.experimental import pallas as pl
from jax.experimental.pallas import tpu as pltpu
from jax.experimental.pallas import tpu_sc as plsc

NC = 2
NS = 16
NW = NC * NS

ROWS = 16384
COLS = 26
DIM = 64
CH = 512
CB = ROWS // CH
NQ = COLS * CB
QW = NQ // NW
assert QW * NW == NQ


def _gather_body(idx_hbm, table_hbm, out_hbm, idx_v, rows_v, sem):
    wid = lax.axis_index("s") * NC + lax.axis_index("c")

    def step(i, carry):
        q = wid * QW + i
        c = q // CB
        b0 = (q % CB) * CH
        pltpu.sync_copy(idx_hbm.at[c, pl.ds(b0, CH)], idx_v)
        pltpu.async_copy(table_hbm.at[idx_v], rows_v, sem).wait()
        pltpu.sync_copy(rows_v, out_hbm.at[c, pl.ds(b0, CH)])
        return carry

    lax.fori_loop(0, QW, step, 0)


@jax.jit
def _gather(idxT, table):
    mesh = plsc.VectorSubcoreMesh(core_axis_name="c", subcore_axis_name="s")
    return pl.kernel(
        _gather_body,
        out_type=jax.ShapeDtypeStruct((COLS, ROWS, DIM), jnp.float32),
        mesh=mesh,
        scratch_types=(
            pltpu.VMEM((CH,), jnp.int32),
            pltpu.VMEM((CH, DIM), jnp.float32),
            pltpu.SemaphoreType.DMA,
        ),
        compiler_params=pltpu.CompilerParams(use_tc_tiling_on_sc=False),
    )(idxT, table)


def kernel(attributes, table):
    idxT = attributes.T.astype(jnp.int32)
    out = _gather(idxT, table)
    return out.transpose(1, 0, 2)


# speedup vs baseline: 1.0892x; 1.0013x over previous
import jax
import jax.numpy as jnp
from jax import lax
from jax.experimental import pallas as pl
from jax.experimental.pallas import tpu as pltpu
from jax.experimental.pallas import tpu_sc as plsc

NC = 2
NS = 16
NW = NC * NS

ROWS = 16384
COLS = 26
DIM = 64
B = ROWS * COLS
CW = ROWS // NW
CH = 512
CB = ROWS // CH
NQ = COLS * CB
QW = NQ // NW


def _format_body(idxT_hbm, out_hbm, idx_v, sems):
    wid = lax.axis_index("s") * NC + lax.axis_index("c")
    col0 = wid * CW
    pltpu.sync_copy(idxT_hbm.at[:, pl.ds(col0, CW)], idx_v)
    descs = [
        pltpu.async_copy(idx_v.at[c], out_hbm.at[pl.ds(c * ROWS + col0, CW)],
                         sems.at[c])
        for c in range(COLS)
    ]
    for d in descs:
        d.wait()


@jax.jit
def _format(idxT):
    mesh = plsc.VectorSubcoreMesh(core_axis_name="c", subcore_axis_name="s")
    return pl.kernel(
        _format_body,
        out_type=jax.ShapeDtypeStruct((B,), jnp.int32),
        mesh=mesh,
        scratch_types=(
            pltpu.VMEM((COLS, CW), jnp.int32),
            pltpu.SemaphoreType.DMA((COLS,)),
        ),
        compiler_params=pltpu.CompilerParams(use_tc_tiling_on_sc=True),
    )(idxT)


def _gather_body(idx_hbm, table_hbm, out_hbm, idx_v, rows_v, sem):
    wid = lax.axis_index("s") * NC + lax.axis_index("c")

    def step(i, carry):
        q = wid * QW + i
        c = q // CB
        b0 = (q % CB) * CH
        pltpu.sync_copy(idx_hbm.at[pl.ds(q * CH, CH)], idx_v)
        pltpu.async_copy(table_hbm.at[idx_v], rows_v, sem).wait()
        pltpu.sync_copy(rows_v, out_hbm.at[c, pl.ds(b0, CH)])
        return carry

    lax.fori_loop(0, QW, step, 0)


@jax.jit
def _gather(idx1d, table):
    mesh = plsc.VectorSubcoreMesh(core_axis_name="c", subcore_axis_name="s")
    return pl.kernel(
        _gather_body,
        out_type=jax.ShapeDtypeStruct((COLS, ROWS, DIM), jnp.float32),
        mesh=mesh,
        scratch_types=(
            pltpu.VMEM((CH,), jnp.int32),
            pltpu.VMEM((CH, DIM), jnp.float32),
            pltpu.SemaphoreType.DMA,
        ),
        compiler_params=pltpu.CompilerParams(use_tc_tiling_on_sc=False),
    )(idx1d, table)


def kernel(attributes, table):
    idxT = attributes.T.astype(jnp.int32)
    idx1d = _format(idxT)
    out = _gather(idx1d, table)
    return out.transpose(1, 0, 2)


# speedup vs baseline: 1.1166x; 1.0252x over previous
import jax
import jax.numpy as jnp
from jax import lax
from jax.experimental import pallas as pl
from jax.experimental.pallas import tpu as pltpu
from jax.experimental.pallas import tpu_sc as plsc

NC = 2
NS = 16
NW = NC * NS

ROWS = 16384
COLS = 26
DIM = 64
B = ROWS * COLS
CW = ROWS // NW
CH = 512
CB = ROWS // CH
NQ = COLS * CB
QW = NQ // NW


def _format_body(idxT_hbm, out_hbm, idx_v, sems):
    wid = lax.axis_index("s") * NC + lax.axis_index("c")
    col0 = wid * CW
    pltpu.sync_copy(idxT_hbm.at[:, pl.ds(col0, CW)], idx_v)
    descs = [
        pltpu.async_copy(idx_v.at[c], out_hbm.at[pl.ds(c * ROWS + col0, CW)],
                         sems.at[c])
        for c in range(COLS)
    ]
    for d in descs:
        d.wait()


@jax.jit
def _format(idxT):
    mesh = plsc.VectorSubcoreMesh(core_axis_name="c", subcore_axis_name="s")
    return pl.kernel(
        _format_body,
        out_type=jax.ShapeDtypeStruct((B,), jnp.int32),
        mesh=mesh,
        scratch_types=(
            pltpu.VMEM((COLS, CW), jnp.int32),
            pltpu.SemaphoreType.DMA((COLS,)),
        ),
        compiler_params=pltpu.CompilerParams(use_tc_tiling_on_sc=True),
    )(idxT)


def _gather_body(idx_hbm, table_hbm, out_hbm, idx_v, rows_v, sem):
    wid = lax.axis_index("s") * NC + lax.axis_index("c")

    def step(i, carry):
        q = wid * QW + i
        c = q // CB
        b0 = (q % CB) * CH
        pltpu.sync_copy(idx_hbm.at[pl.ds(q * CH, CH)], idx_v)
        pltpu.async_copy(table_hbm.at[idx_v], rows_v, sem).wait()
        pltpu.sync_copy(rows_v, out_hbm.at[c, pl.ds(b0, CH), pl.ds(0, DIM)])
        return carry

    lax.fori_loop(0, QW, step, 0)


@jax.jit
def _gather(idx1d, table):
    mesh = plsc.VectorSubcoreMesh(core_axis_name="c", subcore_axis_name="s")
    return pl.kernel(
        _gather_body,
        out_type=jax.ShapeDtypeStruct((COLS, ROWS, 2 * DIM), jnp.float32),
        mesh=mesh,
        scratch_types=(
            pltpu.VMEM((CH,), jnp.int32),
            pltpu.VMEM((CH, DIM), jnp.float32),
            pltpu.SemaphoreType.DMA,
        ),
        compiler_params=pltpu.CompilerParams(use_tc_tiling_on_sc=False),
    )(idx1d, table)


def kernel(attributes, table):
    idxT = attributes.T.astype(jnp.int32)
    idx1d = _format(idxT)
    out = _gather(idx1d, table)
    return out[:, :, :DIM].transpose(1, 0, 2)


# speedup vs baseline: 1.5818x; 1.4167x over previous
import jax
import jax.numpy as jnp
from jax import lax
from jax.experimental import pallas as pl
from jax.experimental.pallas import tpu as pltpu
from jax.experimental.pallas import tpu_sc as plsc

NC = 2
NS = 16
NW = NC * NS

ROWS = 16384
COLS = 26
DIM = 64
B = ROWS * COLS
CW = ROWS // NW
CH = 512
CB = ROWS // CH
NQ = COLS * CB
QW = NQ // NW

MB = 8192
NMB = -(-1000000 // MB)
Z = NMB * MB


def _format_body(idxT_hbm, out_hbm, idx_v, sems):
    wid = lax.axis_index("s") * NC + lax.axis_index("c")
    col0 = wid * CW
    pltpu.sync_copy(idxT_hbm.at[:, pl.ds(col0, CW)], idx_v)
    descs = [
        pltpu.async_copy(idx_v.at[c], out_hbm.at[pl.ds(c * ROWS + col0, CW)],
                         sems.at[c])
        for c in range(COLS)
    ]
    for d in descs:
        d.wait()


@jax.jit
def _format(idxT):
    mesh = plsc.VectorSubcoreMesh(core_axis_name="c", subcore_axis_name="s")
    return pl.kernel(
        _format_body,
        out_type=jax.ShapeDtypeStruct((B,), jnp.int32),
        mesh=mesh,
        scratch_types=(
            pltpu.VMEM((COLS, CW), jnp.int32),
            pltpu.SemaphoreType.DMA((COLS,)),
        ),
        compiler_params=pltpu.CompilerParams(use_tc_tiling_on_sc=True),
    )(idxT)


def _merge_body(in_ref, out_ref):
    x = in_ref[...]
    y = x.T
    out_ref[...] = jnp.concatenate([y, y], axis=1)


@jax.jit
def _merge(tT):
    return pl.pallas_call(
        _merge_body,
        out_shape=jax.ShapeDtypeStruct((Z, 128), jnp.float32),
        grid=(NMB,),
        in_specs=[pl.BlockSpec((64, MB), lambda i: (0, i))],
        out_specs=pl.BlockSpec((MB, 128), lambda i: (i, 0)),
    )(tT)


def _gather_body(idx_hbm, table_hbm, out_hbm, idx_v, rows_v, sem):
    wid = lax.axis_index("s") * NC + lax.axis_index("c")

    def step(i, carry):
        q = wid * QW + i
        c = q // CB
        b0 = (q % CB) * CH
        pltpu.sync_copy(idx_hbm.at[pl.ds(q * CH, CH)], idx_v)

        def dbl(k, carry2):
            v = idx_v[pl.ds(k * 16, 16)]
            idx_v[pl.ds(k * 16, 16)] = v + v
            return carry2

        lax.fori_loop(0, CH // 16, dbl, 0)
        pltpu.async_copy(table_hbm.at[idx_v], rows_v, sem).wait()
        pltpu.sync_copy(rows_v, out_hbm.at[c, pl.ds(b0, CH), pl.ds(0, DIM)])
        return carry

    lax.fori_loop(0, QW, step, 0)


@jax.jit
def _gather(idx1d, table2z):
    mesh = plsc.VectorSubcoreMesh(core_axis_name="c", subcore_axis_name="s")
    return pl.kernel(
        _gather_body,
        out_type=jax.ShapeDtypeStruct((COLS, ROWS, 2 * DIM), jnp.float32),
        mesh=mesh,
        scratch_types=(
            pltpu.VMEM((CH,), jnp.int32),
            pltpu.VMEM((CH, DIM), jnp.float32),
            pltpu.SemaphoreType.DMA,
        ),
        compiler_params=pltpu.CompilerParams(use_tc_tiling_on_sc=False),
    )(idx1d, table2z)


def kernel(attributes, table):
    idxT = attributes.T.astype(jnp.int32)
    idx1d = _format(idxT)
    t2z = _merge(table.T)
    table2z = t2z.reshape(2 * Z, DIM)
    out = _gather(idx1d, table2z)
    return out[:, :, :DIM].transpose(1, 0, 2)


# speedup vs baseline: 1.6256x; 1.0276x over previous
import jax
import jax.numpy as jnp
from jax import lax
from jax.experimental import pallas as pl
from jax.experimental.pallas import tpu as pltpu
from jax.experimental.pallas import tpu_sc as plsc

NC = 2
NS = 16
NW = NC * NS

ROWS = 16384
COLS = 26
DIM = 64
B = ROWS * COLS
CW = ROWS // NW
CH = 1024
CB = ROWS // CH
NQ = COLS * CB
QW = NQ // NW

MB = 8192
NMB = -(-1000000 // MB)
Z = NMB * MB


def _format_body(idxT_hbm, out_hbm, idx_v, sems):
    wid = lax.axis_index("s") * NC + lax.axis_index("c")
    col0 = wid * CW
    pltpu.sync_copy(idxT_hbm.at[:, pl.ds(col0, CW)], idx_v)
    descs = [
        pltpu.async_copy(idx_v.at[c], out_hbm.at[pl.ds(c * ROWS + col0, CW)],
                         sems.at[c])
        for c in range(COLS)
    ]
    for d in descs:
        d.wait()


@jax.jit
def _format(idxT):
    mesh = plsc.VectorSubcoreMesh(core_axis_name="c", subcore_axis_name="s")
    return pl.kernel(
        _format_body,
        out_type=jax.ShapeDtypeStruct((B,), jnp.int32),
        mesh=mesh,
        scratch_types=(
            pltpu.VMEM((COLS, CW), jnp.int32),
            pltpu.SemaphoreType.DMA((COLS,)),
        ),
        compiler_params=pltpu.CompilerParams(use_tc_tiling_on_sc=True),
    )(idxT)


def _merge_body(in_ref, out_ref):
    x = in_ref[...]
    y = x.T
    out_ref[...] = jnp.concatenate([y, y], axis=1)


@jax.jit
def _merge(tT):
    return pl.pallas_call(
        _merge_body,
        out_shape=jax.ShapeDtypeStruct((Z, 128), jnp.float32),
        grid=(NMB,),
        in_specs=[pl.BlockSpec((64, MB), lambda i: (0, i))],
        out_specs=pl.BlockSpec((MB, 128), lambda i: (i, 0)),
    )(tT)


def _gather_body(idx_hbm, table_hbm, out_hbm, idx_v, rows_v, sem):
    wid = lax.axis_index("s") * NC + lax.axis_index("c")

    def step(i, carry):
        q = wid * QW + i
        c = q // CB
        b0 = (q % CB) * CH
        pltpu.sync_copy(idx_hbm.at[pl.ds(q * CH, CH)], idx_v)

        def dbl(k, carry2):
            v = idx_v[pl.ds(k * 16, 16)]
            idx_v[pl.ds(k * 16, 16)] = v + v
            return carry2

        lax.fori_loop(0, CH // 16, dbl, 0)
        pltpu.async_copy(table_hbm.at[idx_v], rows_v, sem).wait()
        pltpu.sync_copy(rows_v, out_hbm.at[c, pl.ds(b0, CH), pl.ds(0, DIM)])
        return carry

    lax.fori_loop(0, QW, step, 0)


@jax.jit
def _gather(idx1d, table2z):
    mesh = plsc.VectorSubcoreMesh(core_axis_name="c", subcore_axis_name="s")
    return pl.kernel(
        _gather_body,
        out_type=jax.ShapeDtypeStruct((COLS, ROWS, 2 * DIM), jnp.float32),
        mesh=mesh,
        scratch_types=(
            pltpu.VMEM((CH,), jnp.int32),
            pltpu.VMEM((CH, DIM), jnp.float32),
            pltpu.SemaphoreType.DMA,
        ),
        compiler_params=pltpu.CompilerParams(use_tc_tiling_on_sc=False),
    )(idx1d, table2z)


def kernel(attributes, table):
    idxT = attributes.T.astype(jnp.int32)
    idx1d = _format(idxT)
    t2z = _merge(table.T)
    table2z = t2z.reshape(2 * Z, DIM)
    out = _gather(idx1d, table2z)
    return out[:, :, :DIM].transpose(1, 0, 2)
